# baseline (speedup 1.0000x reference)
import jax
import jax.numpy as jnp
from jax.experimental import pallas as pl
from jax.experimental.pallas import tpu as pltpu
from jax.experimental.pallas import tpu_sc as plsc

NUM_SLOTS = 100000
KEY_SIZE = 32
MEMORY_SIZE = 128
BATCH = 1024

TILE = 1024
LANES = 128
CHUNKS = TILE // LANES
GRID = (NUM_SLOTS + TILE - 1) // TILE
SEG_ENDS = (32, 65, GRID - 1)

def _argmax_kernel(keys_ref, mk_ref, idx_ref, runv_ref, runi_ref,
                   gbest_ref, gidx_ref):
    i = pl.program_id(0)

    d = jax.lax.dot_general(
        keys_ref[...].astype(jnp.bfloat16), mk_ref[...].astype(jnp.bfloat16),
        dimension_numbers=(((1,), (1,)), ((), ())),
        preferred_element_type=jnp.float32,
    )

    @pl.when(i == 0)
    def _init():
        runv_ref[...] = jnp.full((BATCH, LANES), -jnp.inf, jnp.float32)
        runi_ref[...] = jnp.zeros((BATCH, LANES), jnp.int32)
        gbest_ref[...] = jnp.full((BATCH, 1), -jnp.inf, jnp.float32)
        gidx_ref[...] = jnp.zeros((BATCH, 1), jnp.int32)

    runv = runv_ref[...]
    runi = runi_ref[...]

    lane_iota = jax.lax.broadcasted_iota(jnp.int32, (BATCH, LANES), 1)
    base0 = i * TILE

    def update(runv, runi, masked):
        for c in range(CHUNKS):
            dc = d[:, c * LANES:(c + 1) * LANES]
            base = base0 + c * LANES
            if masked:
                valid = (lane_iota + base) < NUM_SLOTS
                dc = jnp.where(valid, dc, -jnp.inf)
            upd = dc > runv
            runv = jnp.where(upd, dc, runv)
            runi = jnp.where(upd, base, runi)
        return runv, runi

    last = i == GRID - 1

    @pl.when(jnp.logical_not(last))
    def _steady():
        nv, ni = update(runv, runi, masked=False)
        runv_ref[...] = nv
        runi_ref[...] = ni

    @pl.when(last)
    def _masked_tail():
        nv, ni = update(runv, runi, masked=True)
        runv_ref[...] = nv
        runi_ref[...] = ni

    seg_end = (i == SEG_ENDS[0]) | (i == SEG_ENDS[1]) | (i == SEG_ENDS[2])

    @pl.when(seg_end)
    def _fold_segment():
        v = runv_ref[...]
        gi = runi_ref[...] + lane_iota
        m = jnp.max(v, axis=1, keepdims=True)
        cand = jnp.where(v == m, gi, jnp.int32(0x7FFFFFFF))
        seg_idx = jnp.min(cand, axis=1, keepdims=True)
        upd = m > gbest_ref[...]
        mq = m.astype(jnp.bfloat16).astype(jnp.float32)
        gbest_ref[...] = jnp.where(upd, mq, gbest_ref[...])
        gidx_ref[...] = jnp.where(upd, seg_idx, gidx_ref[...])
        runv_ref[...] = jnp.full((BATCH, LANES), -jnp.inf, jnp.float32)
        runi_ref[...] = jnp.zeros((BATCH, LANES), jnp.int32)

    @pl.when(last)
    def _finish():
        idx_ref[...] = gidx_ref[...]


def _closest_slot(keys, memory_keys):
    idx2d = pl.pallas_call(
        _argmax_kernel,
        grid=(GRID,),
        in_specs=[
            pl.BlockSpec((BATCH, KEY_SIZE), lambda i: (0, 0)),
            pl.BlockSpec((TILE, KEY_SIZE), lambda i: (i, 0)),
        ],
        out_specs=pl.BlockSpec((BATCH, 1), lambda i: (0, 0)),
        out_shape=jax.ShapeDtypeStruct((BATCH, 1), jnp.int32),
        scratch_shapes=[
            pltpu.VMEM((BATCH, LANES), jnp.float32),
            pltpu.VMEM((BATCH, LANES), jnp.int32),
            pltpu.VMEM((BATCH, 1), jnp.float32),
            pltpu.VMEM((BATCH, 1), jnp.int32),
        ],
    )(keys, memory_keys)
    return idx2d.reshape(1, BATCH)


_GATHER_WINDOW = 128


def _sc_gather(memory, idx):
    mesh = plsc.VectorSubcoreMesh(core_axis_name="core",
                                  subcore_axis_name="subcore")

    @pl.kernel(out_type=jax.ShapeDtypeStruct((BATCH, MEMORY_SIZE),
                                             memory.dtype),
               mesh=mesh)
    def gather_kernel(mem_hbm, idx_hbm, out_hbm):
        def body(i_vmem, o_vmem):
            pltpu.sync_copy(mem_hbm.at[i_vmem.at[0]], o_vmem)

        pltpu.emit_pipeline(
            body,
            grid=(BATCH // _GATHER_WINDOW,),
            in_specs=[pl.BlockSpec((1, _GATHER_WINDOW),
                                   index_map=lambda i: (0, i))],
            out_specs=[pl.BlockSpec((_GATHER_WINDOW, MEMORY_SIZE),
                                    index_map=lambda i: (i, 0))],
            core_axis_name=("core", "subcore"),
            dimension_semantics=(pltpu.PARALLEL,),
        )(idx_hbm, out_hbm)

    return gather_kernel(memory, idx)


def kernel(keys, memory, memory_keys):
    idx = _closest_slot(keys, memory_keys)
    return _sc_gather(memory, idx)


# speedup vs baseline: 1.1748x; 1.1748x over previous
import jax
import jax.numpy as jnp
from jax.experimental import pallas as pl
from jax.experimental.pallas import tpu as pltpu
from jax.experimental.pallas import tpu_sc as plsc

NUM_SLOTS = 100000
KEY_SIZE = 32
MEMORY_SIZE = 128
BATCH = 1024

TILE = 4224
LANES = 128
CHUNKS = TILE // LANES
GRID = (NUM_SLOTS + TILE - 1) // TILE
SEG_ENDS = (7, 15, GRID - 1)

def _argmax_kernel(keys_ref, mk_ref, idx_ref, runv_ref, runi_ref,
                   gbest_ref, gidx_ref):
    i = pl.program_id(0)

    d = jax.lax.dot_general(
        keys_ref[...].astype(jnp.bfloat16), mk_ref[...].astype(jnp.bfloat16),
        dimension_numbers=(((1,), (1,)), ((), ())),
        preferred_element_type=jnp.float32,
    )

    @pl.when(i == 0)
    def _init():
        runv_ref[...] = jnp.full((BATCH, LANES), -jnp.inf, jnp.float32)
        runi_ref[...] = jnp.zeros((BATCH, LANES), jnp.int32)
        gbest_ref[...] = jnp.full((BATCH, 1), -jnp.inf, jnp.float32)
        gidx_ref[...] = jnp.zeros((BATCH, 1), jnp.int32)

    runv = runv_ref[...]
    runi = runi_ref[...]

    lane_iota = jax.lax.broadcasted_iota(jnp.int32, (BATCH, LANES), 1)
    base0 = i * TILE

    def update(runv, runi, masked):
        for c in range(CHUNKS):
            dc = d[:, c * LANES:(c + 1) * LANES]
            base = base0 + c * LANES
            if masked:
                valid = (lane_iota + base) < NUM_SLOTS
                dc = jnp.where(valid, dc, -jnp.inf)
            upd = dc > runv
            runv = jnp.where(upd, dc, runv)
            runi = jnp.where(upd, base, runi)
        return runv, runi

    last = i == GRID - 1

    @pl.when(jnp.logical_not(last))
    def _steady():
        nv, ni = update(runv, runi, masked=False)
        runv_ref[...] = nv
        runi_ref[...] = ni

    @pl.when(last)
    def _masked_tail():
        nv, ni = update(runv, runi, masked=True)
        runv_ref[...] = nv
        runi_ref[...] = ni

    seg_end = (i == SEG_ENDS[0]) | (i == SEG_ENDS[1]) | (i == SEG_ENDS[2])

    @pl.when(seg_end)
    def _fold_segment():
        v = runv_ref[...]
        gi = runi_ref[...] + lane_iota
        m = jnp.max(v, axis=1, keepdims=True)
        cand = jnp.where(v == m, gi, jnp.int32(0x7FFFFFFF))
        seg_idx = jnp.min(cand, axis=1, keepdims=True)
        upd = m > gbest_ref[...]
        mq = m.astype(jnp.bfloat16).astype(jnp.float32)
        gbest_ref[...] = jnp.where(upd, mq, gbest_ref[...])
        gidx_ref[...] = jnp.where(upd, seg_idx, gidx_ref[...])
        runv_ref[...] = jnp.full((BATCH, LANES), -jnp.inf, jnp.float32)
        runi_ref[...] = jnp.zeros((BATCH, LANES), jnp.int32)

    @pl.when(last)
    def _finish():
        idx_ref[...] = gidx_ref[...]


def _closest_slot(keys, memory_keys):
    idx2d = pl.pallas_call(
        _argmax_kernel,
        grid=(GRID,),
        in_specs=[
            pl.BlockSpec((BATCH, KEY_SIZE), lambda i: (0, 0)),
            pl.BlockSpec((TILE, KEY_SIZE), lambda i: (i, 0)),
        ],
        out_specs=pl.BlockSpec((BATCH, 1), lambda i: (0, 0)),
        out_shape=jax.ShapeDtypeStruct((BATCH, 1), jnp.int32),
        scratch_shapes=[
            pltpu.VMEM((BATCH, LANES), jnp.float32),
            pltpu.VMEM((BATCH, LANES), jnp.int32),
            pltpu.VMEM((BATCH, 1), jnp.float32),
            pltpu.VMEM((BATCH, 1), jnp.int32),
        ],
    )(keys, memory_keys)
    return idx2d.reshape(1, BATCH)


_GATHER_WINDOW = 128


def _sc_gather(memory, idx):
    mesh = plsc.VectorSubcoreMesh(core_axis_name="core",
                                  subcore_axis_name="subcore")

    @pl.kernel(out_type=jax.ShapeDtypeStruct((BATCH, MEMORY_SIZE),
                                             memory.dtype),
               mesh=mesh)
    def gather_kernel(mem_hbm, idx_hbm, out_hbm):
        def body(i_vmem, o_vmem):
            pltpu.sync_copy(mem_hbm.at[i_vmem.at[0]], o_vmem)

        pltpu.emit_pipeline(
            body,
            grid=(BATCH // _GATHER_WINDOW,),
            in_specs=[pl.BlockSpec((1, _GATHER_WINDOW),
                                   index_map=lambda i: (0, i))],
            out_specs=[pl.BlockSpec((_GATHER_WINDOW, MEMORY_SIZE),
                                    index_map=lambda i: (i, 0))],
            core_axis_name=("core", "subcore"),
            dimension_semantics=(pltpu.PARALLEL,),
        )(idx_hbm, out_hbm)

    return gather_kernel(memory, idx)


def kernel(keys, memory, memory_keys):
    idx = _closest_slot(keys, memory_keys)
    return _sc_gather(memory, idx)
